# baseline (speedup 1.0000x reference)
import functools

import jax
import jax.numpy as jnp
from jax import lax
from jax.experimental import pallas as pl
from jax.experimental.pallas import tpu as pltpu
from jax.experimental.pallas import tpu_sc as plsc

N = 10000
NPAD = 10240
E = 320000
EPAD = 327680
CHUNK = 128
NCHS = EPAD // (16 * CHUNK)
CPD = EPAD // (32 * CHUNK)
HALF = 128
BR = 1024
R = NPAD // BR

_mesh = plsc.VectorSubcoreMesh(core_axis_name="c", subcore_axis_name="s")


def _deg_body(col_hbm, ew_hbm, z_hbm, out_hbm, cidx_v, ew_v, buf_v, deg_sh):
    c = lax.axis_index("c")
    s = lax.axis_index("s")
    wid = s * 2 + c
    stripe = s * (NPAD // 16)
    pltpu.sync_copy(z_hbm.at[pl.ds(stripe, NPAD // 16)],
                    deg_sh.at[pl.ds(stripe, NPAD // 16)])
    pltpu.sync_copy(z_hbm.at[pl.ds(0, CHUNK)], buf_v)
    plsc.subcore_barrier()
    dnums = lax.GatherDimensionNumbers(
        offset_dims=(), collapsed_slice_dims=(0,), start_index_map=(0,))

    def chunk(k, _):
        base = (wid * CPD + k) * CHUNK
        pltpu.sync_copy(col_hbm.at[pl.ds(base, CHUNK)], cidx_v)
        pltpu.sync_copy(ew_hbm.at[pl.ds(base, CHUNK)], ew_v)

        def group(g, _):
            ew16 = ew_v[pl.ds(g * 16, 16)]
            for j in range(16):
                ewb = lax.gather(
                    ew16, jnp.full((16, 1), j, jnp.int32), dnums, (1,),
                    mode=lax.GatherScatterMode.PROMISE_IN_BOUNDS)
                e = g * 16 + j
                for f in range(128 // 16):
                    buf_v[e, pl.ds(f * 16, 16)] = (
                        buf_v[e, pl.ds(f * 16, 16)] * 0.0 + ewb)
            return 0

        lax.fori_loop(0, CHUNK // 16, group, 0)
        pltpu.sync_copy(buf_v, deg_sh.at[cidx_v], add=True)
        return 0

    lax.fori_loop(0, CPD, chunk, 0)
    plsc.subcore_barrier()
    pltpu.sync_copy(deg_sh.at[pl.ds(stripe, NPAD // 16)],
                    out_hbm.at[pl.ds(c * NPAD + stripe, NPAD // 16)])


def _make_deg(interpret=False):
    return pl.kernel(
        _deg_body,
        out_type=jax.ShapeDtypeStruct((2 * NPAD, 128), jnp.float32),
        mesh=_mesh,
        scratch_types=[
            pltpu.VMEM((CHUNK,), jnp.int32),
            pltpu.VMEM((CHUNK,), jnp.float32),
            pltpu.VMEM((CHUNK, 128), jnp.float32),
            pltpu.VMEM_SHARED((NPAD, 128), jnp.float32),
        ],
        interpret=interpret,
    )


_deg = _make_deg()


def _prop_body(y_hbm, pk_hbm, ew_hbm, z_hbm, out_hbm, *sc):
    ei = sc[0:4]
    ev = sc[4:8]
    rb = sc[8:10]
    si = sc[10:14]
    sg = sc[14:16]
    ss = sc[16:18]
    acc_sh = sc[18]
    c = lax.axis_index("c")
    s = lax.axis_index("s")
    stripe = s * (NPAD // 16)
    pltpu.sync_copy(z_hbm.at[pl.ds(stripe, NPAD // 16)],
                    acc_sh.at[pl.ds(stripe, NPAD // 16)])
    plsc.subcore_barrier()
    dnums = lax.GatherDimensionNumbers(
        offset_dims=(), collapsed_slice_dims=(0,), start_index_map=(0,))

    def idx_load(k, slot):
        gid = s * NCHS + k
        pltpu.async_copy(pk_hbm.at[pl.ds(gid * 4, 4)], ei[slot], si[slot])
        pltpu.async_copy(ew_hbm.at[pl.ds(gid * CHUNK, CHUNK)], ev[slot],
                         si[slot])

    def idx_wait(slot):
        pltpu.make_async_copy(pk_hbm.at[pl.ds(0, 4)], ei[slot],
                              si[slot]).wait()
        pltpu.make_async_copy(ew_hbm.at[pl.ds(0, CHUNK)], ev[slot],
                              si[slot]).wait()

    def gather(k, slot):
        pltpu.async_copy(y_hbm.at[ei[slot].at[c]], rb[slot % 2],
                         sg[slot % 2])

    def gather_wait(slot):
        pltpu.make_async_copy(y_hbm.at[ei[slot].at[c]], rb[slot % 2],
                              sg[slot % 2]).wait()

    def mul_scatter(b):
        rbs = b % 2

        def group(g, _):
            ew16 = ev[b][pl.ds(g * 16, 16)]
            for j in range(16):
                ewb = lax.gather(
                    ew16, jnp.full((16, 1), j, jnp.int32), dnums, (1,),
                    mode=lax.GatherScatterMode.PROMISE_IN_BOUNDS)
                e = g * 16 + j
                for f in range(HALF // 16):
                    rb[rbs][e, pl.ds(f * 16, 16)] = (
                        rb[rbs][e, pl.ds(f * 16, 16)] * ewb)
            return 0

        lax.fori_loop(0, CHUNK // 16, group, 0)

    def scat_issue(b):
        pltpu.async_copy(rb[b % 2], acc_sh.at[ei[b].at[2]], ss[b % 2],
                         add=True)

    def scat_wait(b):
        pltpu.make_async_copy(rb[b % 2], acc_sh.at[ei[b].at[2]],
                              ss[b % 2]).wait()

    def ops(k, b, w_scat, nxt, nxt2):
        gather_wait(b)
        if w_scat:
            scat_wait((b + 3) % 4)
        if nxt:
            idx_wait((b + 1) % 4)
            gather(k + 1, (b + 1) % 4)
        if nxt2:
            idx_load(k + 2, (b + 2) % 4)
        mul_scatter(b)
        scat_issue(b)

    idx_load(0, 0)
    idx_load(1, 1)
    idx_wait(0)
    gather(0, 0)

    ops(0, 0, False, True, True)
    ops(1, 1, True, True, True)
    ops(2, 2, True, True, True)
    ops(3, 3, True, True, True)

    def quad(kk, _):
        for b in range(4):
            ops(kk * 4 + b, b, True, True, True)
        return 0

    lax.fori_loop(1, NCHS // 4 - 1, quad, 0)
    kl = NCHS - 4
    ops(kl, 0, True, True, True)
    ops(kl + 1, 1, True, True, True)
    ops(kl + 2, 2, True, True, False)
    ops(kl + 3, 3, True, False, False)
    scat_wait(3)
    plsc.subcore_barrier()
    pltpu.sync_copy(acc_sh.at[pl.ds(stripe, NPAD // 16)],
                    out_hbm.at[pl.ds(c * NPAD + stripe, NPAD // 16)])


def _make_prop(interpret=False):
    return pl.kernel(
        _prop_body,
        out_type=jax.ShapeDtypeStruct((2 * NPAD, HALF), jnp.float32),
        mesh=_mesh,
        scratch_types=(
            [pltpu.VMEM((4, CHUNK), jnp.int32)] * 4
            + [pltpu.VMEM((CHUNK,), jnp.float32)] * 4
            + [pltpu.VMEM((CHUNK, HALF), jnp.float32)] * 2
            + [pltpu.SemaphoreType.DMA] * 8
            + [pltpu.VMEM_SHARED((NPAD, HALF), jnp.float32)]
        ),
        interpret=interpret,
    )


_prop = _make_prop()


def _mmxw_body(x_ref, w_ref, xw_ref):
    xw_ref[...] = jnp.dot(x_ref[...], w_ref[...],
                          preferred_element_type=jnp.float32)


def _mmxw(x, w):
    return pl.pallas_call(
        _mmxw_body,
        grid=(R,),
        in_specs=[
            pl.BlockSpec((BR, 128), lambda r: (r, 0)),
            pl.BlockSpec((128, 2 * HALF), lambda r: (0, 0)),
        ],
        out_specs=pl.BlockSpec((BR, 2 * HALF), lambda r: (r, 0)),
        out_shape=jax.ShapeDtypeStruct((NPAD, 2 * HALF), jnp.float32),
    )(x, w)


def _scale_body(xw_ref, deg_ref, y_ref, base_ref):
    xw = xw_ref[...]
    deg = 1.0 + deg_ref[0, :, 0:1] + deg_ref[1, :, 0:1]
    y_ref[0] = xw * lax.rsqrt(deg)
    base_ref[...] = xw / deg


def _scale(xw, degs):
    return pl.pallas_call(
        _scale_body,
        grid=(R, 2),
        in_specs=[
            pl.BlockSpec((BR, HALF), lambda r, h: (r, h)),
            pl.BlockSpec((2, BR, 128), lambda r, h: (0, r, 0)),
        ],
        out_specs=[
            pl.BlockSpec((1, BR, HALF), lambda r, h: (h, r, 0)),
            pl.BlockSpec((BR, HALF), lambda r, h: (r, h)),
        ],
        out_shape=[
            jax.ShapeDtypeStruct((2, NPAD, HALF), jnp.float32),
            jax.ShapeDtypeStruct((NPAD, 2 * HALF), jnp.float32),
        ],
    )(xw, degs)


def _mm2_body(acc_ref, base_ref, w_ref, deg_ref, y_ref, base2_ref):
    deg = 1.0 + deg_ref[0, :, 0:1] + deg_ref[1, :, 0:1]
    dis = lax.rsqrt(deg)
    h = jnp.concatenate([acc_ref[0], acc_ref[1]], axis=1)
    h = jax.nn.relu(base_ref[...] + dis * h)
    xw = jnp.dot(h, w_ref[...], preferred_element_type=jnp.float32)
    y_ref[0] = xw * dis
    base2_ref[...] = xw / deg


def _mm2(acc, base, w, degs):
    return pl.pallas_call(
        _mm2_body,
        grid=(R, 2),
        in_specs=[
            pl.BlockSpec((2, BR, HALF), lambda r, h: (0, r, 0)),
            pl.BlockSpec((BR, 2 * HALF), lambda r, h: (r, 0)),
            pl.BlockSpec((2 * HALF, HALF), lambda r, h: (0, h)),
            pl.BlockSpec((2, BR, 128), lambda r, h: (0, r, 0)),
        ],
        out_specs=[
            pl.BlockSpec((1, BR, HALF), lambda r, h: (h, r, 0)),
            pl.BlockSpec((BR, HALF), lambda r, h: (r, h)),
        ],
        out_shape=[
            jax.ShapeDtypeStruct((2, NPAD, HALF), jnp.float32),
            jax.ShapeDtypeStruct((NPAD, 2 * HALF), jnp.float32),
        ],
    )(acc, base, w, degs)


def _fin_body(acc_ref, base_ref, deg_ref, out_ref):
    deg = 1.0 + deg_ref[0, :, 0:1] + deg_ref[1, :, 0:1]
    dis = lax.rsqrt(deg)
    h = jnp.concatenate([acc_ref[0], acc_ref[1]], axis=1)
    out_ref[...] = jax.nn.relu(base_ref[...] + dis * h)


def _fin(acc, base, degs):
    return pl.pallas_call(
        _fin_body,
        grid=(R,),
        in_specs=[
            pl.BlockSpec((2, BR, HALF), lambda r: (0, r, 0)),
            pl.BlockSpec((BR, 2 * HALF), lambda r: (r, 0)),
            pl.BlockSpec((2, BR, 128), lambda r: (0, r, 0)),
        ],
        out_specs=pl.BlockSpec((BR, 2 * HALF), lambda r: (r, 0)),
        out_shape=jax.ShapeDtypeStruct((NPAD, 2 * HALF), jnp.float32),
    )(acc, base, degs)


def kernel(x, edge_index, edge_weight, W0, W1):
    row = jnp.pad(edge_index[0], (0, EPAD - E))
    col = jnp.pad(edge_index[1], (0, EPAD - E))
    ew = jnp.pad(edge_weight, (0, EPAD - E))
    xp = jnp.pad(x, ((0, NPAD - N), (0, 0)))
    zh = jnp.zeros((NPAD, HALF), jnp.float32)
    rr = row.reshape(-1, CHUNK)
    pk = jnp.stack(
        [rr, rr + NPAD, col.reshape(-1, CHUNK), col.reshape(-1, CHUNK)],
        axis=1)
    pk2 = pk.reshape(-1, CHUNK)

    xw1 = _mmxw(xp, W0)
    degs = _deg(col, ew, zh).reshape(2, NPAD, 128)
    y1, base1 = _scale(xw1, degs)
    acc1 = _prop(y1.reshape(2 * NPAD, HALF), pk2, ew, zh)
    y2, base2 = _mm2(acc1.reshape(2, NPAD, HALF), base1, W1, degs)
    acc2 = _prop(y2.reshape(2 * NPAD, HALF), pk2, ew, zh)
    out = _fin(acc2.reshape(2, NPAD, HALF), base2, degs)
    return out[:N]


# speedup vs baseline: 1.0849x; 1.0849x over previous
import functools

import jax
import jax.numpy as jnp
from jax import lax
from jax.experimental import pallas as pl
from jax.experimental.pallas import tpu as pltpu
from jax.experimental.pallas import tpu_sc as plsc

N = 10000
NPAD = 10240
E = 320000
EPAD = 327680
CHUNK = 128
NCHS = EPAD // (16 * CHUNK)
CPD = EPAD // (32 * CHUNK)
HALF = 128
BR = 1024
R = NPAD // BR

_mesh = plsc.VectorSubcoreMesh(core_axis_name="c", subcore_axis_name="s")


def _deg_body(dk_hbm, z_hbm, out_hbm, dk_v, buf_v, deg_sh):
    c = lax.axis_index("c")
    s = lax.axis_index("s")
    wid = s * 2 + c
    stripe = s * (NPAD // 16)
    pltpu.sync_copy(z_hbm.at[pl.ds(stripe, NPAD // 16)],
                    deg_sh.at[pl.ds(stripe, NPAD // 16)])
    pltpu.sync_copy(z_hbm.at[pl.ds(0, CHUNK)], buf_v)
    plsc.subcore_barrier()
    dnums = lax.GatherDimensionNumbers(
        offset_dims=(), collapsed_slice_dims=(0,), start_index_map=(0,))

    def chunk(k, _):
        gid = wid * CPD + k
        pltpu.sync_copy(dk_hbm.at[pl.ds(gid * 2, 2)], dk_v)

        def group(g, _):
            ew16 = lax.convert_element_type(
                dk_v[1, pl.ds(g * 16, 16)],
                jnp.float32) * 5.9604644775390625e-08
            for j in range(16):
                ewb = lax.gather(
                    ew16, jnp.full((16, 1), j, jnp.int32), dnums, (1,),
                    mode=lax.GatherScatterMode.PROMISE_IN_BOUNDS)
                e = g * 16 + j
                for f in range(128 // 16):
                    buf_v[e, pl.ds(f * 16, 16)] = (
                        buf_v[e, pl.ds(f * 16, 16)] * 0.0 + ewb)
            return 0

        lax.fori_loop(0, CHUNK // 16, group, 0)
        pltpu.sync_copy(buf_v, deg_sh.at[dk_v.at[0]], add=True)
        return 0

    lax.fori_loop(0, CPD, chunk, 0)
    plsc.subcore_barrier()
    pltpu.sync_copy(deg_sh.at[pl.ds(stripe, NPAD // 16)],
                    out_hbm.at[pl.ds(c * NPAD + stripe, NPAD // 16)])


def _make_deg(interpret=False):
    return pl.kernel(
        _deg_body,
        out_type=jax.ShapeDtypeStruct((2 * NPAD, 128), jnp.float32),
        mesh=_mesh,
        scratch_types=[
            pltpu.VMEM((2, CHUNK), jnp.int32),
            pltpu.VMEM((CHUNK, 128), jnp.float32),
            pltpu.VMEM_SHARED((NPAD, 128), jnp.float32),
        ],
        interpret=interpret,
    )


_deg = _make_deg()


def _prop_body(y_hbm, pk_hbm, z_hbm, out_hbm, *sc):
    ei = sc[0:4]
    rb = sc[4:6]
    si = sc[6:10]
    sg = sc[10:12]
    acc_sh = sc[12]
    c = lax.axis_index("c")
    s = lax.axis_index("s")
    stripe = s * (NPAD // 16)
    pltpu.sync_copy(z_hbm.at[pl.ds(stripe, NPAD // 16)],
                    acc_sh.at[pl.ds(stripe, NPAD // 16)])
    plsc.subcore_barrier()
    dnums = lax.GatherDimensionNumbers(
        offset_dims=(), collapsed_slice_dims=(0,), start_index_map=(0,))

    def idx_load(k, slot):
        gid = s * NCHS + k
        pltpu.async_copy(pk_hbm.at[pl.ds(gid * 4, 4)], ei[slot], si[slot])

    def idx_wait(slot):
        pltpu.make_async_copy(pk_hbm.at[pl.ds(0, 4)], ei[slot],
                              si[slot]).wait()

    def gather(k, slot):
        pltpu.async_copy(y_hbm.at[ei[slot].at[c]], rb[slot % 2],
                         sg[slot % 2])

    def gather_wait(slot):
        pltpu.make_async_copy(y_hbm.at[ei[slot].at[c]], rb[slot % 2],
                              sg[slot % 2]).wait()

    def mul_scatter(b):
        rbs = b % 2

        def group(g, _):
            ew16 = lax.convert_element_type(
                ei[b][3, pl.ds(g * 16, 16)],
                jnp.float32) * 5.9604644775390625e-08
            for j in range(16):
                ewb = lax.gather(
                    ew16, jnp.full((16, 1), j, jnp.int32), dnums, (1,),
                    mode=lax.GatherScatterMode.PROMISE_IN_BOUNDS)
                e = g * 16 + j
                for f in range(HALF // 16):
                    rb[rbs][e, pl.ds(f * 16, 16)] = (
                        rb[rbs][e, pl.ds(f * 16, 16)] * ewb)
            return 0

        lax.fori_loop(0, CHUNK // 16, group, 0)
        pltpu.sync_copy(rb[rbs], acc_sh.at[ei[b].at[2]], add=True)

    idx_load(0, 0)
    idx_load(1, 1)
    idx_wait(0)
    gather(0, 0)

    def quad(kk, _):
        for b in range(4):
            k = kk * 4 + b
            gather_wait(b)
            idx_wait((b + 1) % 4)
            gather(k + 1, (b + 1) % 4)
            idx_load(k + 2, (b + 2) % 4)
            mul_scatter(b)
        return 0

    lax.fori_loop(0, NCHS // 4 - 1, quad, 0)
    kl = NCHS - 4
    gather_wait(0)
    idx_wait(1)
    gather(kl + 1, 1)
    idx_load(kl + 2, 2)
    mul_scatter(0)
    gather_wait(1)
    idx_wait(2)
    gather(kl + 2, 2)
    idx_load(kl + 3, 3)
    mul_scatter(1)
    gather_wait(2)
    idx_wait(3)
    gather(kl + 3, 3)
    mul_scatter(2)
    gather_wait(3)
    mul_scatter(3)
    plsc.subcore_barrier()
    pltpu.sync_copy(acc_sh.at[pl.ds(stripe, NPAD // 16)],
                    out_hbm.at[pl.ds(c * NPAD + stripe, NPAD // 16)])


def _make_prop(interpret=False):
    return pl.kernel(
        _prop_body,
        out_type=jax.ShapeDtypeStruct((2 * NPAD, HALF), jnp.float32),
        mesh=_mesh,
        scratch_types=(
            [pltpu.VMEM((4, CHUNK), jnp.int32)] * 4
            + [pltpu.VMEM((CHUNK, HALF), jnp.float32)] * 2
            + [pltpu.SemaphoreType.DMA] * 6
            + [pltpu.VMEM_SHARED((NPAD, HALF), jnp.float32)]
        ),
        interpret=interpret,
    )


_prop = _make_prop()


def _mm1_body(x_ref, w_ref, deg_ref, y_ref, base_ref):
    xw = jnp.dot(x_ref[...], w_ref[...], preferred_element_type=jnp.float32)
    deg = 1.0 + deg_ref[0, :, 0:1] + deg_ref[1, :, 0:1]
    y_ref[0] = xw * lax.rsqrt(deg)
    base_ref[...] = xw / deg


def _mm1(x, w, degs):
    return pl.pallas_call(
        _mm1_body,
        grid=(R, 2),
        in_specs=[
            pl.BlockSpec((BR, 128), lambda r, h: (r, 0)),
            pl.BlockSpec((128, HALF), lambda r, h: (0, h)),
            pl.BlockSpec((2, BR, 128), lambda r, h: (0, r, 0)),
        ],
        out_specs=[
            pl.BlockSpec((1, BR, HALF), lambda r, h: (h, r, 0)),
            pl.BlockSpec((BR, HALF), lambda r, h: (r, h)),
        ],
        out_shape=[
            jax.ShapeDtypeStruct((2, NPAD, HALF), jnp.float32),
            jax.ShapeDtypeStruct((NPAD, 2 * HALF), jnp.float32),
        ],
    )(x, w, degs)


def _mm2_body(acc_ref, base_ref, w_ref, deg_ref, y_ref, base2_ref):
    deg = 1.0 + deg_ref[0, :, 0:1] + deg_ref[1, :, 0:1]
    dis = lax.rsqrt(deg)
    h = jnp.concatenate([acc_ref[0], acc_ref[1]], axis=1)
    h = jax.nn.relu(base_ref[...] + dis * h)
    xw = jnp.dot(h, w_ref[...], preferred_element_type=jnp.float32)
    y_ref[0] = xw * dis
    base2_ref[...] = xw / deg


def _mm2(acc, base, w, degs):
    return pl.pallas_call(
        _mm2_body,
        grid=(R, 2),
        in_specs=[
            pl.BlockSpec((2, BR, HALF), lambda r, h: (0, r, 0)),
            pl.BlockSpec((BR, 2 * HALF), lambda r, h: (r, 0)),
            pl.BlockSpec((2 * HALF, HALF), lambda r, h: (0, h)),
            pl.BlockSpec((2, BR, 128), lambda r, h: (0, r, 0)),
        ],
        out_specs=[
            pl.BlockSpec((1, BR, HALF), lambda r, h: (h, r, 0)),
            pl.BlockSpec((BR, HALF), lambda r, h: (r, h)),
        ],
        out_shape=[
            jax.ShapeDtypeStruct((2, NPAD, HALF), jnp.float32),
            jax.ShapeDtypeStruct((NPAD, 2 * HALF), jnp.float32),
        ],
    )(acc, base, w, degs)


def _fin_body(acc_ref, base_ref, deg_ref, out_ref):
    deg = 1.0 + deg_ref[0, :, 0:1] + deg_ref[1, :, 0:1]
    dis = lax.rsqrt(deg)
    h = jnp.concatenate([acc_ref[0], acc_ref[1]], axis=1)
    out_ref[...] = jax.nn.relu(base_ref[...] + dis * h)


def _fin(acc, base, degs):
    return pl.pallas_call(
        _fin_body,
        grid=(R,),
        in_specs=[
            pl.BlockSpec((2, BR, HALF), lambda r: (0, r, 0)),
            pl.BlockSpec((BR, 2 * HALF), lambda r: (r, 0)),
            pl.BlockSpec((2, BR, 128), lambda r: (0, r, 0)),
        ],
        out_specs=pl.BlockSpec((BR, 2 * HALF), lambda r: (r, 0)),
        out_shape=jax.ShapeDtypeStruct((NPAD, 2 * HALF), jnp.float32),
    )(acc, base, degs)


def kernel(x, edge_index, edge_weight, W0, W1):
    row = jnp.pad(edge_index[0], (0, EPAD - E))
    col = jnp.pad(edge_index[1], (0, EPAD - E))
    ew = jnp.pad(edge_weight, (0, EPAD - E))
    xp = jnp.pad(x, ((0, NPAD - N), (0, 0)))
    zh = jnp.zeros((NPAD, HALF), jnp.float32)
    ewfix = (ew * 16777216.0).astype(jnp.int32)
    rr = row.reshape(-1, CHUNK)
    cc = col.reshape(-1, CHUNK)
    ee = ewfix.reshape(-1, CHUNK)
    pk2 = jnp.stack([rr, rr + NPAD, cc, ee], axis=1).reshape(-1, CHUNK)
    dk2 = jnp.stack([cc, ee], axis=1).reshape(-1, CHUNK)

    degs = _deg(dk2, zh).reshape(2, NPAD, 128)

    y1, base1 = _mm1(xp, W0, degs)
    acc1 = _prop(y1.reshape(2 * NPAD, HALF), pk2, zh)
    y2, base2 = _mm2(acc1.reshape(2, NPAD, HALF), base1, W1, degs)
    acc2 = _prop(y2.reshape(2 * NPAD, HALF), pk2, zh)
    out = _fin(acc2.reshape(2, NPAD, HALF), base2, degs)
    return out[:N]


# speedup vs baseline: 1.0889x; 1.0037x over previous
import functools

import jax
import jax.numpy as jnp
from jax import lax
from jax.experimental import pallas as pl
from jax.experimental.pallas import tpu as pltpu
from jax.experimental.pallas import tpu_sc as plsc

N = 10000
NPAD = 10240
E = 320000
EPAD = 327680
CHUNK = 128
NCHS = EPAD // (16 * CHUNK)
CPD = EPAD // (32 * CHUNK)
HALF = 128
BR = 1024
R = NPAD // BR

_mesh = plsc.VectorSubcoreMesh(core_axis_name="c", subcore_axis_name="s")


def _deg_body(dk_hbm, z_hbm, out_hbm, dk_v, buf_v, deg_sh):
    c = lax.axis_index("c")
    s = lax.axis_index("s")
    wid = s * 2 + c
    stripe = s * (NPAD // 16)
    pltpu.sync_copy(z_hbm.at[pl.ds(stripe, NPAD // 16)],
                    deg_sh.at[pl.ds(stripe, NPAD // 16)])
    pltpu.sync_copy(z_hbm.at[pl.ds(0, CHUNK)], buf_v)
    plsc.subcore_barrier()
    dnums = lax.GatherDimensionNumbers(
        offset_dims=(), collapsed_slice_dims=(0,), start_index_map=(0,))

    def chunk(k, _):
        gid = wid * CPD + k
        pltpu.sync_copy(dk_hbm.at[pl.ds(gid * 2, 2)], dk_v)

        def group(g, _):
            ew16 = lax.convert_element_type(
                dk_v[1, pl.ds(g * 16, 16)],
                jnp.float32) * 5.9604644775390625e-08
            for j in range(16):
                ewb = lax.gather(
                    ew16, jnp.full((16, 1), j, jnp.int32), dnums, (1,),
                    mode=lax.GatherScatterMode.PROMISE_IN_BOUNDS)
                e = g * 16 + j
                for f in range(128 // 16):
                    buf_v[e, pl.ds(f * 16, 16)] = (
                        buf_v[e, pl.ds(f * 16, 16)] * 0.0 + ewb)
            return 0

        for g in range(CHUNK // 16):
            group(g, 0)
        pltpu.sync_copy(buf_v, deg_sh.at[dk_v.at[0]], add=True)
        return 0

    lax.fori_loop(0, CPD, chunk, 0)
    plsc.subcore_barrier()
    pltpu.sync_copy(deg_sh.at[pl.ds(stripe, NPAD // 16)],
                    out_hbm.at[pl.ds(c * NPAD + stripe, NPAD // 16)])


def _make_deg(interpret=False):
    return pl.kernel(
        _deg_body,
        out_type=jax.ShapeDtypeStruct((2 * NPAD, 128), jnp.float32),
        mesh=_mesh,
        scratch_types=[
            pltpu.VMEM((2, CHUNK), jnp.int32),
            pltpu.VMEM((CHUNK, 128), jnp.float32),
            pltpu.VMEM_SHARED((NPAD, 128), jnp.float32),
        ],
        interpret=interpret,
    )


_deg = _make_deg()


def _prop_body(y_hbm, pk_hbm, z_hbm, out_hbm, *sc):
    ei = sc[0:4]
    rb = sc[4:6]
    si = sc[6:10]
    sg = sc[10:12]
    acc_sh = sc[12]
    c = lax.axis_index("c")
    s = lax.axis_index("s")
    stripe = s * (NPAD // 16)
    pltpu.sync_copy(z_hbm.at[pl.ds(stripe, NPAD // 16)],
                    acc_sh.at[pl.ds(stripe, NPAD // 16)])
    plsc.subcore_barrier()
    dnums = lax.GatherDimensionNumbers(
        offset_dims=(), collapsed_slice_dims=(0,), start_index_map=(0,))

    def idx_load(k, slot):
        gid = s * NCHS + k
        pltpu.async_copy(pk_hbm.at[pl.ds(gid * 4, 4)], ei[slot], si[slot])

    def idx_wait(slot):
        pltpu.make_async_copy(pk_hbm.at[pl.ds(0, 4)], ei[slot],
                              si[slot]).wait()

    def gather(k, slot):
        pltpu.async_copy(y_hbm.at[ei[slot].at[c]], rb[slot % 2],
                         sg[slot % 2])

    def gather_wait(slot):
        pltpu.make_async_copy(y_hbm.at[ei[slot].at[c]], rb[slot % 2],
                              sg[slot % 2]).wait()

    def mul_scatter(b):
        rbs = b % 2

        def group(g, _):
            ew16 = lax.convert_element_type(
                ei[b][3, pl.ds(g * 16, 16)],
                jnp.float32) * 5.9604644775390625e-08
            for j in range(16):
                ewb = lax.gather(
                    ew16, jnp.full((16, 1), j, jnp.int32), dnums, (1,),
                    mode=lax.GatherScatterMode.PROMISE_IN_BOUNDS)
                e = g * 16 + j
                for f in range(HALF // 16):
                    rb[rbs][e, pl.ds(f * 16, 16)] = (
                        rb[rbs][e, pl.ds(f * 16, 16)] * ewb)
            return 0

        lax.fori_loop(0, CHUNK // 16, group, 0)
        pltpu.sync_copy(rb[rbs], acc_sh.at[ei[b].at[2]], add=True)

    idx_load(0, 0)
    idx_load(1, 1)
    idx_wait(0)
    gather(0, 0)

    def quad(kk, _):
        for b in range(4):
            k = kk * 4 + b
            gather_wait(b)
            idx_wait((b + 1) % 4)
            gather(k + 1, (b + 1) % 4)
            idx_load(k + 2, (b + 2) % 4)
            mul_scatter(b)
        return 0

    lax.fori_loop(0, NCHS // 4 - 1, quad, 0)
    kl = NCHS - 4
    gather_wait(0)
    idx_wait(1)
    gather(kl + 1, 1)
    idx_load(kl + 2, 2)
    mul_scatter(0)
    gather_wait(1)
    idx_wait(2)
    gather(kl + 2, 2)
    idx_load(kl + 3, 3)
    mul_scatter(1)
    gather_wait(2)
    idx_wait(3)
    gather(kl + 3, 3)
    mul_scatter(2)
    gather_wait(3)
    mul_scatter(3)
    plsc.subcore_barrier()
    pltpu.sync_copy(acc_sh.at[pl.ds(stripe, NPAD // 16)],
                    out_hbm.at[pl.ds(c * NPAD + stripe, NPAD // 16)])


def _make_prop(interpret=False):
    return pl.kernel(
        _prop_body,
        out_type=jax.ShapeDtypeStruct((2 * NPAD, HALF), jnp.float32),
        mesh=_mesh,
        scratch_types=(
            [pltpu.VMEM((4, CHUNK), jnp.int32)] * 4
            + [pltpu.VMEM((CHUNK, HALF), jnp.float32)] * 2
            + [pltpu.SemaphoreType.DMA] * 6
            + [pltpu.VMEM_SHARED((NPAD, HALF), jnp.float32)]
        ),
        interpret=interpret,
    )


_prop = _make_prop()


def _mm1_body(x_ref, w_ref, deg_ref, y_ref, base_ref):
    xw = jnp.dot(x_ref[...], w_ref[...], preferred_element_type=jnp.float32)
    deg = 1.0 + deg_ref[0, :, 0:1] + deg_ref[1, :, 0:1]
    y_ref[0] = xw * lax.rsqrt(deg)
    base_ref[...] = xw / deg


def _mm1(x, w, degs):
    return pl.pallas_call(
        _mm1_body,
        grid=(R, 2),
        in_specs=[
            pl.BlockSpec((BR, 128), lambda r, h: (r, 0)),
            pl.BlockSpec((128, HALF), lambda r, h: (0, h)),
            pl.BlockSpec((2, BR, 128), lambda r, h: (0, r, 0)),
        ],
        out_specs=[
            pl.BlockSpec((1, BR, HALF), lambda r, h: (h, r, 0)),
            pl.BlockSpec((BR, HALF), lambda r, h: (r, h)),
        ],
        out_shape=[
            jax.ShapeDtypeStruct((2, NPAD, HALF), jnp.float32),
            jax.ShapeDtypeStruct((NPAD, 2 * HALF), jnp.float32),
        ],
    )(x, w, degs)


def _mm2_body(acc_ref, base_ref, w_ref, deg_ref, y_ref, base2_ref):
    deg = 1.0 + deg_ref[0, :, 0:1] + deg_ref[1, :, 0:1]
    dis = lax.rsqrt(deg)
    h = jnp.concatenate([acc_ref[0], acc_ref[1]], axis=1)
    h = jax.nn.relu(base_ref[...] + dis * h)
    xw = jnp.dot(h, w_ref[...], preferred_element_type=jnp.float32)
    y_ref[0] = xw * dis
    base2_ref[...] = xw / deg


def _mm2(acc, base, w, degs):
    return pl.pallas_call(
        _mm2_body,
        grid=(R, 2),
        in_specs=[
            pl.BlockSpec((2, BR, HALF), lambda r, h: (0, r, 0)),
            pl.BlockSpec((BR, 2 * HALF), lambda r, h: (r, 0)),
            pl.BlockSpec((2 * HALF, HALF), lambda r, h: (0, h)),
            pl.BlockSpec((2, BR, 128), lambda r, h: (0, r, 0)),
        ],
        out_specs=[
            pl.BlockSpec((1, BR, HALF), lambda r, h: (h, r, 0)),
            pl.BlockSpec((BR, HALF), lambda r, h: (r, h)),
        ],
        out_shape=[
            jax.ShapeDtypeStruct((2, NPAD, HALF), jnp.float32),
            jax.ShapeDtypeStruct((NPAD, 2 * HALF), jnp.float32),
        ],
    )(acc, base, w, degs)


def _fin_body(acc_ref, base_ref, deg_ref, out_ref):
    deg = 1.0 + deg_ref[0, :, 0:1] + deg_ref[1, :, 0:1]
    dis = lax.rsqrt(deg)
    h = jnp.concatenate([acc_ref[0], acc_ref[1]], axis=1)
    out_ref[...] = jax.nn.relu(base_ref[...] + dis * h)


def _fin(acc, base, degs):
    return pl.pallas_call(
        _fin_body,
        grid=(R,),
        in_specs=[
            pl.BlockSpec((2, BR, HALF), lambda r: (0, r, 0)),
            pl.BlockSpec((BR, 2 * HALF), lambda r: (r, 0)),
            pl.BlockSpec((2, BR, 128), lambda r: (0, r, 0)),
        ],
        out_specs=pl.BlockSpec((BR, 2 * HALF), lambda r: (r, 0)),
        out_shape=jax.ShapeDtypeStruct((NPAD, 2 * HALF), jnp.float32),
    )(acc, base, degs)


def kernel(x, edge_index, edge_weight, W0, W1):
    row = jnp.pad(edge_index[0], (0, EPAD - E))
    col = jnp.pad(edge_index[1], (0, EPAD - E))
    ew = jnp.pad(edge_weight, (0, EPAD - E))
    xp = jnp.pad(x, ((0, NPAD - N), (0, 0)))
    zh = jnp.zeros((NPAD, HALF), jnp.float32)
    ewfix = (ew * 16777216.0).astype(jnp.int32)
    rr = row.reshape(-1, CHUNK)
    cc = col.reshape(-1, CHUNK)
    ee = ewfix.reshape(-1, CHUNK)
    pk2 = jnp.stack([rr, rr + NPAD, cc, ee], axis=1).reshape(-1, CHUNK)
    dk2 = jnp.stack([cc, ee], axis=1).reshape(-1, CHUNK)

    degs = _deg(dk2, zh).reshape(2, NPAD, 128)

    y1, base1 = _mm1(xp, W0, degs)
    acc1 = _prop(y1.reshape(2 * NPAD, HALF), pk2, zh)
    y2, base2 = _mm2(acc1.reshape(2, NPAD, HALF), base1, W1, degs)
    acc2 = _prop(y2.reshape(2 * NPAD, HALF), pk2, zh)
    out = _fin(acc2.reshape(2, NPAD, HALF), base2, degs)
    return out[:N]
